# baseline (speedup 1.0000x reference)
import functools

import jax
import jax.numpy as jnp
from jax import lax
from jax.experimental import pallas as pl
from jax.experimental.pallas import tpu as pltpu
from jax.experimental.pallas import tpu_sc as plsc

VOCAB = 1000
HIDDEN = 32
HPAD = 128
B, L = 4096, 20
NTOK = B * L

NC, NS = 2, 16
NW = NC * NS
BPW = NTOK // NW
K = 128
NCH = BPW // K
NBUF = 2

VBLK = 200
NVB = VOCAB // VBLK


def _sc_gather_body(emb_hbm, idx_hbm, h_hbm, idx_v, bufs, sems):
    wid = lax.axis_index("s") * NC + lax.axis_index("c")
    pltpu.sync_copy(idx_hbm.at[pl.ds(wid * NCH, NCH)], idx_v)
    base = wid * BPW

    pltpu.async_copy(emb_hbm.at[idx_v.at[0]], bufs.at[0], sems.at[0])

    def body(g, carry):
        for s in range(NBUF):
            c = g * NBUF + s
            pltpu.make_async_copy(
                emb_hbm.at[idx_v.at[c]], bufs.at[s], sems.at[s]).wait()
            nxt = (s + 1) % NBUF

            @pl.when(c + 1 < NCH)
            def _():
                pltpu.async_copy(
                    emb_hbm.at[idx_v.at[c + 1]], bufs.at[nxt], sems.at[nxt])

            pltpu.sync_copy(bufs.at[s], h_hbm.at[pl.ds(base + c * K, K)])
        return carry

    lax.fori_loop(0, NCH // NBUF, body, 0)


_sc_gather = functools.partial(
    pl.kernel,
    out_type=jax.ShapeDtypeStruct((NTOK, HPAD), jnp.float32),
    mesh=plsc.VectorSubcoreMesh(core_axis_name="c", subcore_axis_name="s"),
    scratch_types=[
        pltpu.VMEM((NCH, K), jnp.int32),
        pltpu.VMEM((NBUF, K, HPAD), jnp.float32),
        pltpu.SemaphoreType.DMA((NBUF,)),
    ],
    compiler_params=pltpu.CompilerParams(use_tc_tiling_on_sc=False),
)(_sc_gather_body)


def _mm_body(w_ref, h_ref, b_ref, out_ref):
    acc = lax.dot_general(
        w_ref[...], h_ref[0],
        dimension_numbers=(((1,), (1,)), ((), ())),
        preferred_element_type=jnp.float32,
    ) + b_ref[...]
    out_ref[...] = acc[None]


_mm_call = pl.pallas_call(
    _mm_body,
    grid=(L, NVB),
    in_specs=[
        pl.BlockSpec((VBLK, HPAD), lambda l, vb: (vb, 0)),
        pl.BlockSpec((1, B, HPAD), lambda l, vb: (l, 0, 0)),
        pl.BlockSpec((VBLK, 1), lambda l, vb: (vb, 0)),
    ],
    out_specs=pl.BlockSpec((1, VBLK, B), lambda l, vb: (l, vb, 0)),
    out_shape=jax.ShapeDtypeStruct((L, VOCAB, B), jnp.float32),
    compiler_params=pltpu.CompilerParams(
        dimension_semantics=("arbitrary", "arbitrary"),
    ),
)


@jax.jit
def kernel(x, emb_table, W, b):
    emb_pad = jnp.zeros((VOCAB, HPAD), jnp.float32).at[:, :HIDDEN].set(emb_table)
    w_pad = jnp.zeros((VOCAB, HPAD), jnp.float32).at[:, :HIDDEN].set(W)
    idx = x.T.reshape(NTOK // K, K)
    h = _sc_gather(emb_pad, idx)
    h3 = h.reshape(L, B, HPAD)
    out_phys = _mm_call(w_pad, h3, b.reshape(VOCAB, 1))
    return out_phys.transpose(2, 0, 1)


# speedup vs baseline: 1.2605x; 1.2605x over previous
import functools

import jax
import jax.numpy as jnp
from jax import lax
from jax.experimental import pallas as pl
from jax.experimental.pallas import tpu as pltpu
from jax.experimental.pallas import tpu_sc as plsc

VOCAB = 1000
HIDDEN = 32
HPAD = 128
B, L = 4096, 20
NTOK = B * L

NC, NS = 2, 16
NW = NC * NS
BPW = NTOK // NW
K = 128
NCH = BPW // K
NBUF = 2

BLK_B = 4096
NBB = B // BLK_B


def _sc_gather_body(emb_hbm, idx_hbm, h_hbm, idx_v, bufs, sems):
    wid = lax.axis_index("s") * NC + lax.axis_index("c")
    pltpu.sync_copy(idx_hbm.at[pl.ds(wid * NCH, NCH)], idx_v)
    base = wid * BPW

    pltpu.async_copy(emb_hbm.at[idx_v.at[0]], bufs.at[0], sems.at[0])

    def body(g, carry):
        for s in range(NBUF):
            c = g * NBUF + s
            pltpu.make_async_copy(
                emb_hbm.at[idx_v.at[c]], bufs.at[s], sems.at[s]).wait()
            nxt = (s + 1) % NBUF

            @pl.when(c + 1 < NCH)
            def _():
                pltpu.async_copy(
                    emb_hbm.at[idx_v.at[c + 1]], bufs.at[nxt], sems.at[nxt])

            pltpu.sync_copy(bufs.at[s], h_hbm.at[pl.ds(base + c * K, K)])
        return carry

    lax.fori_loop(0, NCH // NBUF, body, 0)


_sc_gather = functools.partial(
    pl.kernel,
    out_type=jax.ShapeDtypeStruct((NTOK, HPAD), jnp.float32),
    mesh=plsc.VectorSubcoreMesh(core_axis_name="c", subcore_axis_name="s"),
    scratch_types=[
        pltpu.VMEM((NCH, K), jnp.int32),
        pltpu.VMEM((NBUF, K, HPAD), jnp.float32),
        pltpu.SemaphoreType.DMA((NBUF,)),
    ],
    compiler_params=pltpu.CompilerParams(use_tc_tiling_on_sc=False),
)(_sc_gather_body)


def _mm_body(w_ref, h_ref, b_ref, out_ref):
    acc = lax.dot_general(
        w_ref[...], h_ref[...],
        dimension_numbers=(((1,), (1,)), ((), ())),
        preferred_element_type=jnp.float32,
    ) + b_ref[...]
    out_ref[...] = acc[None]


_mm_call = pl.pallas_call(
    _mm_body,
    grid=(L, NBB),
    in_specs=[
        pl.BlockSpec((VOCAB, HPAD), lambda l, bb: (0, 0)),
        pl.BlockSpec((BLK_B, HPAD), lambda l, bb: (l * NBB + bb, 0)),
        pl.BlockSpec((VOCAB, 1), lambda l, bb: (0, 0)),
    ],
    out_specs=pl.BlockSpec((1, VOCAB, BLK_B), lambda l, bb: (l, 0, bb)),
    out_shape=jax.ShapeDtypeStruct((L, VOCAB, B), jnp.float32),
    compiler_params=pltpu.CompilerParams(
        dimension_semantics=("arbitrary", "arbitrary"),
        vmem_limit_bytes=60 * 1024 * 1024,
    ),
)


@jax.jit
def kernel(x, emb_table, W, b):
    emb_pad = jnp.zeros((VOCAB, HPAD), jnp.float32).at[:, :HIDDEN].set(emb_table)
    w_pad = jnp.zeros((VOCAB, HPAD), jnp.float32).at[:, :HIDDEN].set(W)
    idx = x.T.reshape(NTOK // K, K)
    h = _sc_gather(emb_pad, idx)
    out_phys = _mm_call(w_pad, h, b.reshape(VOCAB, 1))
    return out_phys.transpose(2, 0, 1)
